# baseline (speedup 1.0000x reference)
import jax
import jax.numpy as jnp
from jax.experimental import pallas as pl

_N = 10000
_D = 64
_K = 32
_ALPHA = 3.0
_NP = 10240
_R = 200
_NB = _N // _R


def _mlp_body(e1_ref, e2_ref, w1_ref, b1_ref, w2_ref, b2_ref, n1_ref, n2_ref):
    dn = (((1,), (1,)), ((), ()))
    h1 = jax.lax.dot_general(e1_ref[...], w1_ref[...], dn)
    h2 = jax.lax.dot_general(e2_ref[...], w2_ref[...], dn)
    n1_ref[...] = jnp.tanh(_ALPHA * (h1 + b1_ref[...]))
    n2_ref[...] = jnp.tanh(_ALPHA * (h2 + b2_ref[...]))


def _adj_body(x1_ref, x2_ref, n1_ref, n2_ref, out_ref):
    dn = (((1,), (1,)), ((), ()))
    a = (jax.lax.dot_general(x1_ref[...], n2_ref[...], dn)
         - jax.lax.dot_general(x2_ref[...], n1_ref[...], dn))
    act = jnp.maximum(jnp.tanh(_ALPHA * a), 0.0)
    vi = jax.lax.bitcast_convert_type(act, jnp.int32)

    nch = _NP // 128
    gi = jnp.max(vi.reshape(_R, nch, 128), axis=2)

    gmax = jnp.max(gi, axis=1, keepdims=True)

    def gstep(_, carry):
        lo, hi = carry
        mid = lo + jax.lax.div(hi - lo, 2)
        cnt = jnp.sum((gi > mid).astype(jnp.int32), axis=1, keepdims=True)
        ge = cnt >= _K
        return jnp.where(ge, mid, lo), jnp.where(ge, hi, mid)

    _, lb = jax.lax.fori_loop(0, 31, gstep,
                              (jnp.full_like(gmax, -1), gmax))

    def vcond(carry):
        lo, hi, _ = carry
        return jnp.any(hi - lo > 1)

    def vstep(carry):
        lo, hi, nhi = carry
        mid = lo + jax.lax.div(hi - lo, 2)
        cnt = jnp.sum((vi > mid).astype(jnp.int32), axis=1, keepdims=True)
        ge = cnt >= _K
        return (jnp.where(ge, mid, lo),
                jnp.where(ge, hi, mid),
                jnp.where(ge, nhi, cnt))

    _, thr, ngt = jax.lax.while_loop(
        vcond, vstep, (lb - 1, gmax, jnp.zeros_like(gmax)))

    need = _K - ngt
    eq3 = (vi.reshape(_R, nch, 128) == thr[:, :, None]).astype(jnp.float32)
    ec = jnp.sum(eq3, axis=2)
    ciota = jax.lax.broadcasted_iota(jnp.int32, (_R, nch), 1)
    needf = need.astype(jnp.float32)

    def chstep(_, carry):
        lo_c, hi_c = carry
        mid = lo_c + jax.lax.div(hi_c - lo_c, 2)
        cnt = jnp.sum(jnp.where(ciota < mid, ec, 0.0), axis=1, keepdims=True)
        ge = cnt >= needf
        return jnp.where(ge, lo_c, mid), jnp.where(ge, mid, hi_c)

    nbits = max(1, (nch - 1).bit_length())
    _, cstar = jax.lax.fori_loop(0, nbits + 1, chstep,
                                 (jnp.zeros_like(gmax),
                                  jnp.full_like(gmax, nch)))
    cut_ch = cstar - 1
    before = jnp.sum(jnp.where(ciota < cut_ch, ec, 0.0), axis=1,
                     keepdims=True)
    resid = needf - before
    onehot = (ciota == cut_ch).astype(jnp.float32)
    eqch = jnp.sum(eq3 * onehot[:, :, None], axis=1)
    liota = jax.lax.broadcasted_iota(jnp.int32, (_R, 128), 1)

    def lstep(_, carry):
        lo_c, hi_c = carry
        mid = lo_c + jax.lax.div(hi_c - lo_c, 2)
        cnt = jnp.sum(jnp.where(liota < mid, eqch, 0.0), axis=1,
                      keepdims=True)
        ge = cnt >= resid
        return jnp.where(ge, lo_c, mid), jnp.where(ge, mid, hi_c)

    _, tstar = jax.lax.fori_loop(0, 8, lstep,
                                 (jnp.zeros_like(gmax),
                                  jnp.full_like(gmax, 128)))
    cut = cut_ch * 128 + tstar

    cols = jax.lax.broadcasted_iota(jnp.int32, (_R, _NP), 1)
    keep = (vi > thr) | ((vi == thr) & (cols < cut))
    out_ref[...] = jnp.where(keep, act, 0.0)[:, :_N]


def kernel(idx, emb1, emb2, W1, b1, W2, b2):
    e1 = jnp.take(emb1, idx, axis=0)
    e2 = jnp.take(emb2, idx, axis=0)
    pad = ((0, _NP - _N), (0, 0))
    e1p = jnp.pad(e1, pad)
    e2p = jnp.pad(e2, pad)
    nv_shape = jax.ShapeDtypeStruct((_NP, _D), jnp.float32)
    n1p, n2p = pl.pallas_call(
        _mlp_body,
        out_shape=[nv_shape, nv_shape],
    )(e1p, e2p, W1, b1.reshape(1, _D), W2, b2.reshape(1, _D))

    row_spec = pl.BlockSpec((_R, _D), lambda i: (i, 0))
    full_spec = pl.BlockSpec((_NP, _D), lambda i: (0, 0))
    adj = pl.pallas_call(
        _adj_body,
        grid=(_NB,),
        in_specs=[row_spec, row_spec, full_spec, full_spec],
        out_specs=pl.BlockSpec((_R, _N), lambda i: (i, 0)),
        out_shape=jax.ShapeDtypeStruct((_N, _N), jnp.float32),
    )(n1p, n2p, n1p, n2p)
    return adj


# speedup vs baseline: 2.6441x; 2.6441x over previous
import jax
import jax.numpy as jnp
from jax.experimental import pallas as pl

_N = 10000
_D = 64
_K = 32
_ALPHA = 3.0
_NP = 10240
_R = 200
_NB = _N // _R


def _mlp_body(e1_ref, e2_ref, w1_ref, b1_ref, w2_ref, b2_ref, n1_ref, n2_ref):
    dn = (((1,), (1,)), ((), ()))
    h1 = jax.lax.dot_general(e1_ref[...], w1_ref[...], dn)
    h2 = jax.lax.dot_general(e2_ref[...], w2_ref[...], dn)
    n1_ref[...] = jnp.tanh(_ALPHA * (h1 + b1_ref[...]))
    n2_ref[...] = jnp.tanh(_ALPHA * (h2 + b2_ref[...]))


def _adj_body(x1_ref, x2_ref, n1_ref, n2_ref, out_ref):
    dn = (((1,), (1,)), ((), ()))
    a = (jax.lax.dot_general(x1_ref[...], n2_ref[...], dn)
         - jax.lax.dot_general(x2_ref[...], n1_ref[...], dn))
    act = jnp.maximum(jnp.tanh(_ALPHA * a), 0.0)
    vi = jax.lax.bitcast_convert_type(act, jnp.int32)
    vi = jnp.maximum(vi, 0)

    m = vi
    w = _NP
    while w > 128 and (w // 2) % 128 == 0:
        m = jnp.maximum(m[:, :w // 2], m[:, w // 2:])
        w //= 2
    gi = m[:, :128]
    for i in range(1, w // 128):
        gi = jnp.maximum(gi, m[:, i * 128:(i + 1) * 128])

    gmax = jnp.max(gi, axis=1, keepdims=True)

    def gstep(_, carry):
        lo, hi = carry
        mid = lo + jax.lax.div(hi - lo, 2)
        cnt = jnp.sum((gi > mid).astype(jnp.int32), axis=1, keepdims=True)
        ge = cnt >= _K
        return jnp.where(ge, mid, lo), jnp.where(ge, hi, mid)

    _, lb = jax.lax.fori_loop(0, 31, gstep,
                              (jnp.full_like(gmax, -1), gmax))

    def vcond(carry):
        lo, hi, _ = carry
        return jnp.any(hi - lo > 1)

    def vstep(carry):
        lo, hi, nhi = carry
        mid = lo + jax.lax.div(hi - lo, 2)
        cnt = jnp.sum((vi > mid).astype(jnp.int32), axis=1, keepdims=True)
        ge = cnt >= _K
        return (jnp.where(ge, mid, lo),
                jnp.where(ge, hi, mid),
                jnp.where(ge, nhi, cnt))

    _, thr, ngt = jax.lax.while_loop(
        vcond, vstep, (lb - 1, gmax, jnp.zeros_like(gmax)))

    need = _K - ngt
    eq = vi == thr
    cols = jax.lax.broadcasted_iota(jnp.int32, (_R, _NP), 1)

    def cstep(_, carry):
        lo_c, hi_c = carry
        mid = lo_c + jax.lax.div(hi_c - lo_c, 2)
        cnt = jnp.sum((eq & (cols < mid)).astype(jnp.int32), axis=1,
                      keepdims=True)
        ge = cnt >= need
        return jnp.where(ge, lo_c, mid), jnp.where(ge, mid, hi_c)

    _, cut = jax.lax.fori_loop(0, 14, cstep,
                               (jnp.zeros_like(gmax),
                                jnp.full_like(gmax, 16384)))

    keep = (vi > thr) | (eq & (cols < cut))
    out_ref[...] = jnp.where(keep, act, 0.0)[:, :_N]


def kernel(idx, emb1, emb2, W1, b1, W2, b2):
    e1 = jnp.take(emb1, idx, axis=0)
    e2 = jnp.take(emb2, idx, axis=0)
    pad = ((0, _NP - _N), (0, 0))
    e1p = jnp.pad(e1, pad)
    e2p = jnp.pad(e2, pad)
    nv_shape = jax.ShapeDtypeStruct((_NP, _D), jnp.float32)
    n1p, n2p = pl.pallas_call(
        _mlp_body,
        out_shape=[nv_shape, nv_shape],
    )(e1p, e2p, W1, b1.reshape(1, _D), W2, b2.reshape(1, _D))

    row_spec = pl.BlockSpec((_R, _D), lambda i: (i, 0))
    full_spec = pl.BlockSpec((_NP, _D), lambda i: (0, 0))
    adj = pl.pallas_call(
        _adj_body,
        grid=(_NB,),
        in_specs=[row_spec, row_spec, full_spec, full_spec],
        out_specs=pl.BlockSpec((_R, _N), lambda i: (i, 0)),
        out_shape=jax.ShapeDtypeStruct((_N, _N), jnp.float32),
    )(n1p, n2p, n1p, n2p)
    return adj


# speedup vs baseline: 2.7241x; 1.0303x over previous
import jax
import jax.numpy as jnp
from jax.experimental import pallas as pl

_N = 10000
_D = 64
_K = 32
_ALPHA = 3.0
_NP = 10240
_R = 200
_NB = _N // _R


def _mlp_body(e1_ref, e2_ref, w1_ref, b1_ref, w2_ref, b2_ref, n1_ref, n2_ref):
    dn = (((1,), (1,)), ((), ()))
    h1 = jax.lax.dot_general(e1_ref[...], w1_ref[...], dn)
    h2 = jax.lax.dot_general(e2_ref[...], w2_ref[...], dn)
    n1_ref[...] = jnp.tanh(_ALPHA * (h1 + b1_ref[...]))
    n2_ref[...] = jnp.tanh(_ALPHA * (h2 + b2_ref[...]))


def _count16(mask16):
    s = mask16
    w = s.shape[1]
    while w > 128 and (w // 2) % 128 == 0:
        s = s[:, :w // 2] + s[:, w // 2:]
        w //= 2
    t = s[:, :128]
    for i in range(1, w // 128):
        t = t + s[:, i * 128:(i + 1) * 128]
    return jnp.sum(t.astype(jnp.int32), axis=1, keepdims=True)


def _adj_body(x1_ref, x2_ref, n1_ref, n2_ref, out_ref):
    dn = (((1,), (1,)), ((), ()))
    a = (jax.lax.dot_general(x1_ref[...], n2_ref[...], dn)
         - jax.lax.dot_general(x2_ref[...], n1_ref[...], dn))
    act = jnp.maximum(jnp.tanh(_ALPHA * a), 0.0)
    vi = jax.lax.bitcast_convert_type(act, jnp.int32)
    vi = jnp.maximum(vi, 0)

    m = vi
    w = _NP
    while w > 128 and (w // 2) % 128 == 0:
        m = jnp.maximum(m[:, :w // 2], m[:, w // 2:])
        w //= 2
    gi = m[:, :128]
    for i in range(1, w // 128):
        gi = jnp.maximum(gi, m[:, i * 128:(i + 1) * 128])

    gmax = jnp.max(gi, axis=1, keepdims=True)

    def gstep(_, carry):
        lo, hi = carry
        mid = lo + jax.lax.div(hi - lo, 2)
        cnt = jnp.sum((gi > mid).astype(jnp.int32), axis=1, keepdims=True)
        ge = cnt >= _K
        return jnp.where(ge, mid, lo), jnp.where(ge, hi, mid)

    _, lb = jax.lax.fori_loop(0, 31, gstep,
                              (jnp.full_like(gmax, -1), gmax))

    base = lb - 1
    v16 = jnp.clip(vi - base, 0, 32767).astype(jnp.int16)
    hi16 = jnp.clip(gmax - base, 0, 32767)

    def vcond16(carry):
        lo, hi, _ = carry
        return jnp.any((hi - lo) > 1)

    def vstep16(carry):
        lo, hi, nhi = carry
        mid = lo + jax.lax.div(hi - lo, 2)
        cnt = _count16(jnp.where(v16 > mid.astype(jnp.int16),
                                 jnp.int16(1), jnp.int16(0)))
        ge = cnt >= _K
        return (jnp.where(ge, mid, lo),
                jnp.where(ge, hi, mid),
                jnp.where(ge, nhi, cnt))

    _, thr16, ngt16 = jax.lax.while_loop(
        vcond16, vstep16,
        (jnp.zeros_like(hi16), hi16, jnp.zeros((_R, 1), jnp.int32)))

    sat = thr16 == 32767
    thr0 = base + thr16
    lo0 = jnp.where(sat, base + 32766, thr0 - 1)
    hiv0 = jnp.where(sat, gmax, thr0)
    nhi0 = jnp.where(sat, 0, ngt16)

    def vcond(carry):
        lo, hi, _ = carry
        return jnp.any(hi - lo > 1)

    def vstep(carry):
        lo, hi, nhi = carry
        mid = lo + jax.lax.div(hi - lo, 2)
        cnt = jnp.sum((vi > mid).astype(jnp.int32), axis=1, keepdims=True)
        ge = cnt >= _K
        return (jnp.where(ge, mid, lo),
                jnp.where(ge, hi, mid),
                jnp.where(ge, nhi, cnt))

    _, thr, ngt = jax.lax.while_loop(vcond, vstep, (lo0, hiv0, nhi0))

    need = _K - ngt
    eq = vi == thr
    eq16 = (jnp.clip(vi - thr + 1, 0, 1)
            - jnp.clip(vi - thr, 0, 1)).astype(jnp.int16)
    cols16 = jax.lax.broadcasted_iota(jnp.int16, (_R, _NP), 1)

    def cstep(_, carry):
        lo_c, hi_c = carry
        mid = lo_c + jax.lax.div(hi_c - lo_c, 2)
        cnt = _count16(jnp.where(cols16 < mid.astype(jnp.int16),
                                 eq16, jnp.int16(0)))
        ge = cnt >= need
        return jnp.where(ge, lo_c, mid), jnp.where(ge, mid, hi_c)

    _, cut = jax.lax.fori_loop(0, 14, cstep,
                               (jnp.zeros_like(gmax),
                                jnp.full_like(gmax, 16384)))

    cols = jax.lax.broadcasted_iota(jnp.int32, (_R, _NP), 1)
    keep = (vi > thr) | (eq & (cols < cut))
    out_ref[...] = jnp.where(keep, act, 0.0)[:, :_N]


def kernel(idx, emb1, emb2, W1, b1, W2, b2):
    e1 = jnp.take(emb1, idx, axis=0)
    e2 = jnp.take(emb2, idx, axis=0)
    pad = ((0, _NP - _N), (0, 0))
    e1p = jnp.pad(e1, pad)
    e2p = jnp.pad(e2, pad)
    nv_shape = jax.ShapeDtypeStruct((_NP, _D), jnp.float32)
    n1p, n2p = pl.pallas_call(
        _mlp_body,
        out_shape=[nv_shape, nv_shape],
    )(e1p, e2p, W1, b1.reshape(1, _D), W2, b2.reshape(1, _D))

    row_spec = pl.BlockSpec((_R, _D), lambda i: (i, 0))
    full_spec = pl.BlockSpec((_NP, _D), lambda i: (0, 0))
    adj = pl.pallas_call(
        _adj_body,
        grid=(_NB,),
        in_specs=[row_spec, row_spec, full_spec, full_spec],
        out_specs=pl.BlockSpec((_R, _N), lambda i: (i, 0)),
        out_shape=jax.ShapeDtypeStruct((_N, _N), jnp.float32),
    )(n1p, n2p, n1p, n2p)
    return adj
